# baseline (speedup 1.0000x reference)
import functools

import jax
import jax.numpy as jnp
from jax import lax
from jax.experimental import pallas as pl
from jax.experimental.pallas import tpu as pltpu
from jax.experimental.pallas import tpu_sc as plsc

_K = 128
_TEMP = 0.7
_TB = 128
_NC = 2
_NS = 16
_L = 16
_NW = _NC * _NS


def _norm_body(x_ref, an_ref, ar_ref):
    x = x_ref[...]
    n = jnp.sqrt(jnp.sum(x * x, axis=1, keepdims=True))
    an_ref[...] = (x / jnp.maximum(n, 1e-12)).astype(jnp.bfloat16)
    ar_ref[...] = x.astype(jnp.bfloat16).astype(jnp.float32)


def _route_body(base_ref, wq_ref, bq_ref, an_ref, w_ref):
    q = jax.lax.dot_general(
        base_ref[...].astype(jnp.bfloat16), wq_ref[...].astype(jnp.bfloat16),
        (((1,), (1,)), ((), ())),
        preferred_element_type=jnp.float32) + bq_ref[...]
    n = jnp.sqrt(jnp.sum(q * q, axis=1, keepdims=True))
    q = q / jnp.maximum(n, 1e-12)
    s = jax.lax.dot_general(
        q.astype(jnp.bfloat16), an_ref[...],
        (((1,), (1,)), ((), ())),
        preferred_element_type=jnp.float32)
    key = jax.lax.bitcast_convert_type(s, jnp.int32)
    key = key ^ (jax.lax.shift_right_arithmetic(key, 31) & jnp.int32(0x7FFFFFFF))

    def bit_step(i, t):
        cand = t + jax.lax.shift_left(jnp.int32(1), jnp.int32(31) - i)
        cnt = jnp.sum((key >= cand).astype(jnp.int32), axis=1, keepdims=True)
        return jnp.where(cnt >= _K, cand, t)

    t0 = jnp.full((s.shape[0], 1), jnp.int32(-2147483647) - 1)
    t = jax.lax.fori_loop(0, 32, bit_step, t0)

    mask = key >= t
    m = jnp.max(s, axis=1, keepdims=True)
    e = jnp.where(mask, jnp.exp((s - m) / _TEMP), 0.0)
    w = e / jnp.sum(e, axis=1, keepdims=True)
    w_ref[...] = w.astype(jnp.bfloat16).astype(jnp.float32)


def _sc_recon_body(A, D, RPW, w_hbm, atom_hbm, out_hbm,
                   w_rowA, w_rowB, idx_bA, idx_bB, wv_bA, wv_bB,
                   idx_gA, idx_gB, rows_bA, rows_bB, acc_b, semA, semB):
    nj = D // _L
    wid = lax.axis_index("s") * _NC + lax.axis_index("c")
    row0 = wid * RPW

    def prepare(r, w_row, idx_b, wv_b, idx_g, rows_b, sem):
        pltpu.sync_copy(w_hbm.at[row0 + r], w_row)

        @plsc.parallel_loop(0, A // _L, 1, unroll=16,
                            carry=jnp.zeros((_L,), jnp.int32))
        def _scan(ci, cnt_v):
            wv = w_row[pl.ds(ci * _L, _L)]
            msk = wv > 0.0
            idxv = lax.iota(jnp.int32, _L) + ci * _L
            ones = jnp.where(msk, jnp.int32(1), jnp.int32(0))
            ranks = plsc.cumsum(ones) - 1 + cnt_v
            plsc.store_scatter(idx_b, [ranks], idxv, mask=msk)
            plsc.store_scatter(wv_b, [ranks], wv, mask=msk)
            return cnt_v + plsc.all_reduce_population_count(msk)

        for j in range(_K // _L):
            idx_g[pl.ds(j * _L, _L)] = idx_b[pl.ds(j * _L, _L)]

        return pltpu.async_copy(atom_hbm.at[idx_g], rows_b, sem)

    def consume(r, wv_b, rows_b):
        accs0 = tuple(jnp.zeros((_L,), jnp.float32) for _ in range(nj))

        @plsc.parallel_loop(0, _K, 1, unroll=8, carry=accs0)
        def accs(k, accs_c):
            wk = plsc.load_gather(wv_b, [jnp.full((_L,), 0, jnp.int32) + k])
            return tuple(accs_c[j] + wk * rows_b[k, pl.ds(j * _L, _L)]
                         for j in range(nj))
        for j in range(nj):
            acc_b[pl.ds(j * _L, _L)] = accs[j]
        pltpu.sync_copy(acc_b, out_hbm.at[row0 + r])

    def waitA():
        pltpu.make_async_copy(atom_hbm.at[idx_gA], rows_bA, semA).wait()

    def waitB():
        pltpu.make_async_copy(atom_hbm.at[idx_gB], rows_bB, semB).wait()

    prepare(0, w_rowA, idx_bA, wv_bA, idx_gA, rows_bA, semA)

    def pair_fn(i, carry):
        r = 2 * i
        prepare(r + 1, w_rowB, idx_bB, wv_bB, idx_gB, rows_bB, semB)
        waitA()
        consume(r, wv_bA, rows_bA)
        prepare(r + 2, w_rowA, idx_bA, wv_bA, idx_gA, rows_bA, semA)
        waitB()
        consume(r + 1, wv_bB, rows_bB)
        return carry

    lax.fori_loop(0, RPW // 2 - 1, pair_fn, jnp.int32(0))

    prepare(RPW - 1, w_rowB, idx_bB, wv_bB, idx_gB, rows_bB, semB)
    waitA()
    consume(RPW - 2, wv_bA, rows_bA)
    waitB()
    consume(RPW - 1, wv_bB, rows_bB)


def _build(B, D, A, interpret=False):
    norm = pl.pallas_call(
        _norm_body,
        grid=(A // 1024,),
        in_specs=[pl.BlockSpec((1024, D), lambda i: (i, 0))],
        out_specs=[pl.BlockSpec((1024, D), lambda i: (i, 0)),
                   pl.BlockSpec((1024, D), lambda i: (i, 0))],
        out_shape=[jax.ShapeDtypeStruct((A, D), jnp.bfloat16),
                   jax.ShapeDtypeStruct((A, D), jnp.float32)],
        interpret=interpret,
    )
    route = pl.pallas_call(
        _route_body,
        grid=(B // _TB,),
        in_specs=[
            pl.BlockSpec((_TB, D), lambda i: (i, 0)),
            pl.BlockSpec((D, D), lambda i: (0, 0)),
            pl.BlockSpec((1, D), lambda i: (0, 0)),
            pl.BlockSpec((A, D), lambda i: (0, 0)),
        ],
        out_specs=pl.BlockSpec((_TB, A), lambda i: (i, 0)),
        out_shape=jax.ShapeDtypeStruct((B, A), jnp.float32),
        interpret=interpret,
    )
    RPW = B // _NW
    mesh = plsc.VectorSubcoreMesh(core_axis_name="c", subcore_axis_name="s")
    recon = pl.kernel(
        functools.partial(_sc_recon_body, A, D, RPW),
        out_type=jax.ShapeDtypeStruct((B, D), jnp.float32),
        mesh=mesh,
        compiler_params=pltpu.CompilerParams(needs_layout_passes=False),
        scratch_types=[
            pltpu.VMEM((A,), jnp.float32),
            pltpu.VMEM((A,), jnp.float32),
            pltpu.VMEM((_K + 2 * _L,), jnp.int32),
            pltpu.VMEM((_K + 2 * _L,), jnp.int32),
            pltpu.VMEM((_K + 2 * _L,), jnp.float32),
            pltpu.VMEM((_K + 2 * _L,), jnp.float32),
            pltpu.VMEM((_K,), jnp.int32),
            pltpu.VMEM((_K,), jnp.int32),
            pltpu.VMEM((_K, D), jnp.float32),
            pltpu.VMEM((_K, D), jnp.float32),
            pltpu.VMEM((D,), jnp.float32),
            pltpu.SemaphoreType.DMA,
            pltpu.SemaphoreType.DMA,
        ],
    )
    return norm, route, recon


def kernel(base_raw, Wq, bq, atom_bank):
    B, D = base_raw.shape
    A = atom_bank.shape[0]
    nchunks = 4
    Bc = B // nchunks
    norm, route, recon = _build(Bc, D, A)
    an, ar = norm(atom_bank)
    outs = []
    for c in range(nchunks):
        chunk = jax.lax.slice_in_dim(base_raw, c * Bc, (c + 1) * Bc, axis=0)
        w = route(chunk, Wq, bq.reshape(1, D), an)
        outs.append(recon(w, ar))
    return jnp.concatenate(outs, axis=0)


# speedup vs baseline: 1.4368x; 1.4368x over previous
import functools

import jax
import jax.numpy as jnp
from jax import lax
from jax.experimental import pallas as pl
from jax.experimental.pallas import tpu as pltpu
from jax.experimental.pallas import tpu_sc as plsc

_K = 128
_TEMP = 0.7
_TB = 128
_NC = 2
_NS = 16
_L = 16
_NW = _NC * _NS


def _norm_body(x_ref, an_ref, ar_ref):
    x = x_ref[...]
    n = jnp.sqrt(jnp.sum(x * x, axis=1, keepdims=True))
    an_ref[...] = (x / jnp.maximum(n, 1e-12)).astype(jnp.bfloat16)
    ar_ref[...] = x.astype(jnp.bfloat16).astype(jnp.float32)


def _route_body(base_ref, wq_ref, bq_ref, an_ref, w_ref):
    q = jax.lax.dot_general(
        base_ref[...].astype(jnp.bfloat16), wq_ref[...].astype(jnp.bfloat16),
        (((1,), (1,)), ((), ())),
        preferred_element_type=jnp.float32) + bq_ref[...]
    n = jnp.sqrt(jnp.sum(q * q, axis=1, keepdims=True))
    q = q / jnp.maximum(n, 1e-12)
    s = jax.lax.dot_general(
        q.astype(jnp.bfloat16), an_ref[...],
        (((1,), (1,)), ((), ())),
        preferred_element_type=jnp.float32)
    key = jax.lax.bitcast_convert_type(s, jnp.int32)
    key = key ^ (jax.lax.shift_right_arithmetic(key, 31) & jnp.int32(0x7FFFFFFF))

    def bit_step(i, t):
        cand = t + jax.lax.shift_left(jnp.int32(1), jnp.int32(31) - i)
        cnt = jnp.sum((key >= cand).astype(jnp.int32), axis=1, keepdims=True)
        return jnp.where(cnt >= _K, cand, t)

    t0 = jnp.full((s.shape[0], 1), jnp.int32(-2147483647) - 1)
    t = jax.lax.fori_loop(0, 32, bit_step, t0)

    mask = key >= t
    m = jnp.max(s, axis=1, keepdims=True)
    e = jnp.where(mask, jnp.exp((s - m) / _TEMP), 0.0)
    w = e / jnp.sum(e, axis=1, keepdims=True)
    w_ref[...] = w.astype(jnp.bfloat16).astype(jnp.float32)


def _sc_recon_body(A, D, RPW, w_hbm, atom_hbm, out_hbm,
                   w_rowA, w_rowB, idx_bA, idx_bB, wv_bA, wv_bB,
                   idx_gA, idx_gB, rows_bA, rows_bB, acc_b, semA, semB):
    nj = D // _L
    wid = lax.axis_index("s") * _NC + lax.axis_index("c")
    row0 = wid * RPW

    def prepare(r, w_row, idx_b, wv_b, idx_g, rows_b, sem):
        pltpu.sync_copy(w_hbm.at[row0 + r], w_row)

        @plsc.parallel_loop(0, A // _L, 1, unroll=8,
                            carry=jnp.zeros((_L,), jnp.int32))
        def _scan(ci, cnt_v):
            wv = w_row[pl.ds(ci * _L, _L)]
            msk = wv > 0.0
            idxv = lax.iota(jnp.int32, _L) + ci * _L
            ones = jnp.where(msk, jnp.int32(1), jnp.int32(0))
            ranks = plsc.cumsum(ones) - 1 + cnt_v
            plsc.store_scatter(idx_b, [ranks], idxv, mask=msk)
            plsc.store_scatter(wv_b, [ranks], wv, mask=msk)
            return cnt_v + plsc.all_reduce_population_count(msk)

        for j in range(_K // _L):
            idx_g[pl.ds(j * _L, _L)] = idx_b[pl.ds(j * _L, _L)]

        return pltpu.async_copy(atom_hbm.at[idx_g], rows_b, sem)

    def consume(r, wv_b, rows_b):
        accs0 = tuple(jnp.zeros((_L,), jnp.float32) for _ in range(nj))

        @plsc.parallel_loop(0, _K, 1, unroll=4, carry=accs0)
        def accs(k, accs_c):
            wk = plsc.load_gather(wv_b, [jnp.full((_L,), 0, jnp.int32) + k])
            return tuple(accs_c[j] + wk * rows_b[k, pl.ds(j * _L, _L)]
                         for j in range(nj))
        for j in range(nj):
            acc_b[pl.ds(j * _L, _L)] = accs[j]
        pltpu.sync_copy(acc_b, out_hbm.at[row0 + r])

    def waitA():
        pltpu.make_async_copy(atom_hbm.at[idx_gA], rows_bA, semA).wait()

    def waitB():
        pltpu.make_async_copy(atom_hbm.at[idx_gB], rows_bB, semB).wait()

    prepare(0, w_rowA, idx_bA, wv_bA, idx_gA, rows_bA, semA)

    def pair_fn(i, carry):
        r = 2 * i
        prepare(r + 1, w_rowB, idx_bB, wv_bB, idx_gB, rows_bB, semB)
        waitA()
        consume(r, wv_bA, rows_bA)
        prepare(r + 2, w_rowA, idx_bA, wv_bA, idx_gA, rows_bA, semA)
        waitB()
        consume(r + 1, wv_bB, rows_bB)
        return carry

    lax.fori_loop(0, RPW // 2 - 1, pair_fn, jnp.int32(0))

    prepare(RPW - 1, w_rowB, idx_bB, wv_bB, idx_gB, rows_bB, semB)
    waitA()
    consume(RPW - 2, wv_bA, rows_bA)
    waitB()
    consume(RPW - 1, wv_bB, rows_bB)


def _build(B, D, A, interpret=False):
    norm = pl.pallas_call(
        _norm_body,
        grid=(A // 1024,),
        in_specs=[pl.BlockSpec((1024, D), lambda i: (i, 0))],
        out_specs=[pl.BlockSpec((1024, D), lambda i: (i, 0)),
                   pl.BlockSpec((1024, D), lambda i: (i, 0))],
        out_shape=[jax.ShapeDtypeStruct((A, D), jnp.bfloat16),
                   jax.ShapeDtypeStruct((A, D), jnp.float32)],
        interpret=interpret,
    )
    route = pl.pallas_call(
        _route_body,
        grid=(B // _TB,),
        in_specs=[
            pl.BlockSpec((_TB, D), lambda i: (i, 0)),
            pl.BlockSpec((D, D), lambda i: (0, 0)),
            pl.BlockSpec((1, D), lambda i: (0, 0)),
            pl.BlockSpec((A, D), lambda i: (0, 0)),
        ],
        out_specs=pl.BlockSpec((_TB, A), lambda i: (i, 0)),
        out_shape=jax.ShapeDtypeStruct((B, A), jnp.float32),
        interpret=interpret,
    )
    RPW = B // _NW
    mesh = plsc.VectorSubcoreMesh(core_axis_name="c", subcore_axis_name="s")
    recon = pl.kernel(
        functools.partial(_sc_recon_body, A, D, RPW),
        out_type=jax.ShapeDtypeStruct((B, D), jnp.float32),
        mesh=mesh,
        compiler_params=pltpu.CompilerParams(needs_layout_passes=False),
        scratch_types=[
            pltpu.VMEM((A,), jnp.float32),
            pltpu.VMEM((A,), jnp.float32),
            pltpu.VMEM((_K + 2 * _L,), jnp.int32),
            pltpu.VMEM((_K + 2 * _L,), jnp.int32),
            pltpu.VMEM((_K + 2 * _L,), jnp.float32),
            pltpu.VMEM((_K + 2 * _L,), jnp.float32),
            pltpu.VMEM((_K,), jnp.int32),
            pltpu.VMEM((_K,), jnp.int32),
            pltpu.VMEM((_K, D), jnp.float32),
            pltpu.VMEM((_K, D), jnp.float32),
            pltpu.VMEM((D,), jnp.float32),
            pltpu.SemaphoreType.DMA,
            pltpu.SemaphoreType.DMA,
        ],
    )
    return norm, route, recon


def kernel(base_raw, Wq, bq, atom_bank):
    B, D = base_raw.shape
    A = atom_bank.shape[0]
    nchunks = 4
    Bc = B // nchunks
    norm, route, recon = _build(Bc, D, A)
    an, ar = norm(atom_bank)
    outs = []
    for c in range(nchunks):
        chunk = jax.lax.slice_in_dim(base_raw, c * Bc, (c + 1) * Bc, axis=0)
        w = route(chunk, Wq, bq.reshape(1, D), an)
        outs.append(recon(w, ar))
    return jnp.concatenate(outs, axis=0)


# speedup vs baseline: 1.5380x; 1.0704x over previous
import functools

import jax
import jax.numpy as jnp
from jax import lax
from jax.experimental import pallas as pl
from jax.experimental.pallas import tpu as pltpu
from jax.experimental.pallas import tpu_sc as plsc

_K = 128
_TEMP = 0.7
_TB = 128
_NC = 2
_NS = 16
_L = 16
_NW = _NC * _NS


def _norm_body(x_ref, an_ref, ar_ref):
    x = x_ref[...]
    n = jnp.sqrt(jnp.sum(x * x, axis=1, keepdims=True))
    an_ref[...] = (x / jnp.maximum(n, 1e-12)).astype(jnp.bfloat16)
    ar_ref[...] = x.astype(jnp.bfloat16).astype(jnp.float32)


def _route_body(base_ref, wq_ref, bq_ref, an_ref, w_ref):
    q = jax.lax.dot_general(
        base_ref[...].astype(jnp.bfloat16), wq_ref[...].astype(jnp.bfloat16),
        (((1,), (1,)), ((), ())),
        preferred_element_type=jnp.float32) + bq_ref[...]
    n = jnp.sqrt(jnp.sum(q * q, axis=1, keepdims=True))
    q = q / jnp.maximum(n, 1e-12)
    s = jax.lax.dot_general(
        q.astype(jnp.bfloat16), an_ref[...],
        (((1,), (1,)), ((), ())),
        preferred_element_type=jnp.float32)
    key = jax.lax.bitcast_convert_type(s, jnp.int32)
    key = key ^ (jax.lax.shift_right_arithmetic(key, 31) & jnp.int32(0x7FFFFFFF))

    def bit_step(i, t):
        cand = t + jax.lax.shift_left(jnp.int32(1), jnp.int32(31) - i)
        cnt = jnp.sum((key >= cand).astype(jnp.int32), axis=1, keepdims=True)
        return jnp.where(cnt >= _K, cand, t)

    t0 = jnp.full((s.shape[0], 1), jnp.int32(-2147483647) - 1)
    t = jax.lax.fori_loop(0, 32, bit_step, t0)

    mask = key >= t
    m = jnp.max(s, axis=1, keepdims=True)
    e = jnp.where(mask, jnp.exp((s - m) / _TEMP), 0.0)
    w = e / jnp.sum(e, axis=1, keepdims=True)
    w_ref[...] = w.astype(jnp.bfloat16).astype(jnp.float32)


def _sc_recon_body(A, D, RPW, w_hbm, atom_hbm, out_hbm,
                   w_rowA, w_rowB, idx_bA, idx_bB, wv_bA, wv_bB,
                   idx_gA, idx_gB, rows_bA, rows_bB, acc_b, semA, semB):
    nj = D // _L
    wid = lax.axis_index("s") * _NC + lax.axis_index("c")
    row0 = wid * RPW

    def prepare(r, w_row, idx_b, wv_b, idx_g, rows_b, sem):
        pltpu.sync_copy(w_hbm.at[row0 + r], w_row)

        @plsc.parallel_loop(0, A // _L, 1, unroll=8,
                            carry=jnp.zeros((_L,), jnp.int32))
        def _scan(ci, cnt_v):
            wv = w_row[pl.ds(ci * _L, _L)]
            msk = wv > 0.0
            idxv = lax.iota(jnp.int32, _L) + ci * _L
            ones = jnp.where(msk, jnp.int32(1), jnp.int32(0))
            ranks = plsc.cumsum(ones) - 1 + cnt_v
            plsc.store_scatter(idx_b, [ranks], idxv, mask=msk)
            plsc.store_scatter(wv_b, [ranks], wv, mask=msk)
            return cnt_v + plsc.all_reduce_population_count(msk)

        for j in range(_K // _L):
            idx_g[pl.ds(j * _L, _L)] = idx_b[pl.ds(j * _L, _L)]

        return pltpu.async_copy(atom_hbm.at[idx_g], rows_b, sem)

    def consume(r, wv_b, rows_b):
        accs0 = tuple(jnp.zeros((_L,), jnp.float32) for _ in range(nj))

        @plsc.parallel_loop(0, _K, 1, unroll=4, carry=accs0)
        def accs(k, accs_c):
            wk = plsc.load_gather(wv_b, [jnp.full((_L,), 0, jnp.int32) + k])
            return tuple(accs_c[j] + wk * rows_b[k, pl.ds(j * _L, _L)]
                         for j in range(nj))
        for j in range(nj):
            acc_b[pl.ds(j * _L, _L)] = accs[j]
        pltpu.sync_copy(acc_b, out_hbm.at[row0 + r])

    def waitA():
        pltpu.make_async_copy(atom_hbm.at[idx_gA], rows_bA, semA).wait()

    def waitB():
        pltpu.make_async_copy(atom_hbm.at[idx_gB], rows_bB, semB).wait()

    prepare(0, w_rowA, idx_bA, wv_bA, idx_gA, rows_bA, semA)

    def pair_fn(i, carry):
        r = 2 * i
        prepare(r + 1, w_rowB, idx_bB, wv_bB, idx_gB, rows_bB, semB)
        waitA()
        consume(r, wv_bA, rows_bA)
        prepare(r + 2, w_rowA, idx_bA, wv_bA, idx_gA, rows_bA, semA)
        waitB()
        consume(r + 1, wv_bB, rows_bB)
        return carry

    lax.fori_loop(0, RPW // 2 - 1, pair_fn, jnp.int32(0))

    prepare(RPW - 1, w_rowB, idx_bB, wv_bB, idx_gB, rows_bB, semB)
    waitA()
    consume(RPW - 2, wv_bA, rows_bA)
    waitB()
    consume(RPW - 1, wv_bB, rows_bB)


def _build(B, D, A, interpret=False):
    norm = pl.pallas_call(
        _norm_body,
        grid=(A // 1024,),
        in_specs=[pl.BlockSpec((1024, D), lambda i: (i, 0))],
        out_specs=[pl.BlockSpec((1024, D), lambda i: (i, 0)),
                   pl.BlockSpec((1024, D), lambda i: (i, 0))],
        out_shape=[jax.ShapeDtypeStruct((A, D), jnp.bfloat16),
                   jax.ShapeDtypeStruct((A, D), jnp.float32)],
        interpret=interpret,
    )
    route = pl.pallas_call(
        _route_body,
        grid=(B // _TB,),
        in_specs=[
            pl.BlockSpec((_TB, D), lambda i: (i, 0)),
            pl.BlockSpec((D, D), lambda i: (0, 0)),
            pl.BlockSpec((1, D), lambda i: (0, 0)),
            pl.BlockSpec((A, D), lambda i: (0, 0)),
        ],
        out_specs=pl.BlockSpec((_TB, A), lambda i: (i, 0)),
        out_shape=jax.ShapeDtypeStruct((B, A), jnp.float32),
        interpret=interpret,
    )
    RPW = B // _NW
    mesh = plsc.VectorSubcoreMesh(core_axis_name="c", subcore_axis_name="s")
    recon = pl.kernel(
        functools.partial(_sc_recon_body, A, D, RPW),
        out_type=jax.ShapeDtypeStruct((B, D), jnp.float32),
        mesh=mesh,
        compiler_params=pltpu.CompilerParams(needs_layout_passes=False),
        scratch_types=[
            pltpu.VMEM((A,), jnp.float32),
            pltpu.VMEM((A,), jnp.float32),
            pltpu.VMEM((_K + 2 * _L,), jnp.int32),
            pltpu.VMEM((_K + 2 * _L,), jnp.int32),
            pltpu.VMEM((_K + 2 * _L,), jnp.float32),
            pltpu.VMEM((_K + 2 * _L,), jnp.float32),
            pltpu.VMEM((_K,), jnp.int32),
            pltpu.VMEM((_K,), jnp.int32),
            pltpu.VMEM((_K, D), jnp.float32),
            pltpu.VMEM((_K, D), jnp.float32),
            pltpu.VMEM((D,), jnp.float32),
            pltpu.SemaphoreType.DMA,
            pltpu.SemaphoreType.DMA,
        ],
    )
    return norm, route, recon


def kernel(base_raw, Wq, bq, atom_bank):
    B, D = base_raw.shape
    A = atom_bank.shape[0]
    nchunks = 8
    Bc = B // nchunks
    norm, route, recon = _build(Bc, D, A)
    an, ar = norm(atom_bank)
    outs = []
    for c in range(nchunks):
        chunk = jax.lax.slice_in_dim(base_raw, c * Bc, (c + 1) * Bc, axis=0)
        w = route(chunk, Wq, bq.reshape(1, D), an)
        outs.append(recon(w, ar))
    return jnp.concatenate(outs, axis=0)
